# Initial kernel scaffold; baseline (speedup 1.0000x reference)
import jax
import jax.numpy as jnp
from jax.experimental import pallas as pl


def kernel(x, edge_index, adj_norm_sp, W, bias):
    raise NotImplementedError("write your pallas kernel here")


# speedup vs baseline: 14.5001x; 14.5001x over previous
import functools

import jax
import jax.numpy as jnp
from jax import lax
from jax.experimental import pallas as pl
from jax.experimental.pallas import tpu as pltpu
from jax.experimental.pallas import tpu_sc as plsc

N_NODES = 10000
D = 128
E = 320000

NC = 2
NS = 16
NW = NC * NS
L = 16

CHUNK = 128
NCHUNK = 79
EPW = CHUNK * NCHUNK
E_PAD = EPW * NW
N_ACC = 10240
DUMMY = N_NODES
RPT = N_ACC // NS
GRID = 10
BLK = N_ACC // GRID


def _doctor(r_buf, c_buf):
    for j in range(CHUNK // L):
        sl = pl.ds(j * L, L)
        r = r_buf[sl]
        c = c_buf[sl]
        c_buf[sl] = jnp.where(r == c, jnp.full((L,), DUMMY, jnp.int32), c)


def _deg_body(rowp, colp, out, r_buf, c_buf, ones_b, zbuf, acc, sem):
    core = lax.axis_index("c")
    sid = lax.axis_index("s")
    wid = core * NS + sid

    for j in range(CHUNK // L):
        ones_b[pl.ds(j * L, L)] = jnp.full((L,), 1.0, jnp.float32)
    for j in range(RPT // L):
        zbuf[pl.ds(j * L, L)] = jnp.zeros((L,), jnp.float32)
    pltpu.sync_copy(zbuf, acc.at[pl.ds(sid * RPT, RPT)])
    plsc.subcore_barrier()

    def step(k, carry):
        base = wid * EPW + k * CHUNK
        pltpu.sync_copy(rowp.at[pl.ds(base, CHUNK)], r_buf)
        pltpu.sync_copy(colp.at[pl.ds(base, CHUNK)], c_buf)
        _doctor(r_buf, c_buf)
        pltpu.sync_copy(ones_b, acc.at[c_buf], add=True)
        return carry

    lax.fori_loop(0, NCHUNK, step, 0)
    plsc.subcore_barrier()
    pltpu.sync_copy(acc.at[pl.ds(sid * RPT, RPT)],
                    out.at[core, pl.ds(sid * RPT, RPT)])


_deg_kernel = functools.partial(
    pl.kernel,
    out_type=jax.ShapeDtypeStruct((NC, N_ACC), jnp.float32),
    mesh=plsc.VectorSubcoreMesh(core_axis_name="c", subcore_axis_name="s"),
    scratch_types=[
        pltpu.VMEM((CHUNK,), jnp.int32),
        pltpu.VMEM((CHUNK,), jnp.int32),
        pltpu.VMEM((CHUNK,), jnp.float32),
        pltpu.VMEM((RPT,), jnp.float32),
        pltpu.VMEM_SHARED((N_ACC,), jnp.float32),
        pltpu.SemaphoreType.DMA,
    ],
)(_deg_body)


def _edge_body(rowp, colp, g, out, r_buf, c_buf, vals, zbuf, acc, sem):
    core = lax.axis_index("c")
    sid = lax.axis_index("s")
    wid = core * NS + sid

    for i in range(16):
        for j in range(D // L):
            zbuf[i, pl.ds(j * L, L)] = jnp.zeros((L,), jnp.float32)

    def zstep(k, carry):
        pltpu.sync_copy(zbuf, acc.at[pl.ds(sid * RPT + k * 16, 16)])
        return carry

    lax.fori_loop(0, RPT // 16, zstep, 0)
    plsc.subcore_barrier()

    def step(k, carry):
        base = wid * EPW + k * CHUNK
        pltpu.sync_copy(rowp.at[pl.ds(base, CHUNK)], r_buf)
        pltpu.sync_copy(colp.at[pl.ds(base, CHUNK)], c_buf)
        _doctor(r_buf, c_buf)
        pltpu.async_copy(g.at[r_buf], vals, sem).wait()
        pltpu.sync_copy(vals, acc.at[c_buf], add=True)
        return carry

    lax.fori_loop(0, NCHUNK, step, 0)
    plsc.subcore_barrier()
    pltpu.sync_copy(acc.at[pl.ds(sid * RPT, RPT)],
                    out.at[core, pl.ds(sid * RPT, RPT)])


_edge_kernel = functools.partial(
    pl.kernel,
    out_type=jax.ShapeDtypeStruct((NC, N_ACC, D), jnp.float32),
    mesh=plsc.VectorSubcoreMesh(core_axis_name="c", subcore_axis_name="s"),
    scratch_types=[
        pltpu.VMEM((CHUNK,), jnp.int32),
        pltpu.VMEM((CHUNK,), jnp.int32),
        pltpu.VMEM((CHUNK, D), jnp.float32),
        pltpu.VMEM((16, D), jnp.float32),
        pltpu.VMEM_SHARED((N_ACC, D), jnp.float32),
        pltpu.SemaphoreType.DMA,
    ],
)(_edge_body)


def _mm_body(x_ref, w_ref, deg_ref, g_ref):
    deg = deg_ref[0, :] + deg_ref[1, :] + 1.0
    dis = lax.rsqrt(deg)
    h = lax.dot_general(x_ref[...], w_ref[...], (((1,), (1,)), ((), ())),
                        preferred_element_type=jnp.float32)
    g_ref[...] = h * dis[:, None]


def _comb_body(p_ref, g_ref, deg_ref, b_ref, o_ref):
    deg = deg_ref[0, :] + deg_ref[1, :] + 1.0
    dis = lax.rsqrt(deg)
    s = p_ref[0] + p_ref[1] + g_ref[...]
    o_ref[...] = s * dis[:, None] + b_ref[...]


def kernel(x, edge_index, adj_norm_sp, W, bias):
    row = edge_index[0]
    col = edge_index[1]
    pad = E_PAD - E
    rowp = jnp.concatenate([row, jnp.zeros((pad,), jnp.int32)])
    colp = jnp.concatenate([col, jnp.full((pad,), DUMMY, jnp.int32)])
    x_pad = jnp.concatenate(
        [x, jnp.zeros((N_ACC - N_NODES, D), jnp.float32)], axis=0)
    bias2d = bias.reshape(1, D)

    degp = _deg_kernel(rowp, colp)

    g = pl.pallas_call(
        _mm_body,
        grid=(GRID,),
        in_specs=[
            pl.BlockSpec((BLK, D), lambda i: (i, 0)),
            pl.BlockSpec((D, D), lambda i: (0, 0)),
            pl.BlockSpec((NC, BLK), lambda i: (0, i)),
        ],
        out_specs=pl.BlockSpec((BLK, D), lambda i: (i, 0)),
        out_shape=jax.ShapeDtypeStruct((N_ACC, D), jnp.float32),
    )(x_pad, W, degp)

    parts = _edge_kernel(rowp, colp, g)

    out = pl.pallas_call(
        _comb_body,
        grid=(GRID,),
        in_specs=[
            pl.BlockSpec((NC, BLK, D), lambda i: (0, i, 0)),
            pl.BlockSpec((BLK, D), lambda i: (i, 0)),
            pl.BlockSpec((NC, BLK), lambda i: (0, i)),
            pl.BlockSpec((1, D), lambda i: (0, 0)),
        ],
        out_specs=pl.BlockSpec((BLK, D), lambda i: (i, 0)),
        out_shape=jax.ShapeDtypeStruct((N_ACC, D), jnp.float32),
    )(parts, g, degp, bias2d)

    return out[:N_NODES]


# speedup vs baseline: 14.9198x; 1.0289x over previous
import functools

import jax
import jax.numpy as jnp
from jax import lax
from jax.experimental import pallas as pl
from jax.experimental.pallas import tpu as pltpu
from jax.experimental.pallas import tpu_sc as plsc

N_NODES = 10000
D = 128
E = 320000

NC = 2
NS = 16
NW = NC * NS
L = 16

CHUNK = 128
NCHUNK = 80
EPW = CHUNK * NCHUNK
E_PAD = EPW * NW
N_ACC = 10240
DUMMY = N_NODES
RPT = N_ACC // NS
GRID = 10
BLK = N_ACC // GRID
G = 16
NG = NCHUNK // G
DEG_WIN = 8


def _doctor_group(r_grp, c_grp, gb):
    def doc(k, carry):
        for j in range(CHUNK // L):
            sl = pl.ds(j * L, L)
            r = r_grp[gb, k, sl]
            c = c_grp[gb, k, sl]
            c_grp[gb, k, sl] = jnp.where(
                r == c, jnp.full((L,), DUMMY, jnp.int32), c)
        return carry

    lax.fori_loop(0, G, doc, 0)


def _deg_body(rowp2, colp2, out, r_all, c_all, ones_b, zbuf, acc, sem):
    core = lax.axis_index("c")
    sid = lax.axis_index("s")
    wid = core * NS + sid

    for j in range(CHUNK // L):
        ones_b[pl.ds(j * L, L)] = jnp.full((L,), 1.0, jnp.float32)
    for j in range(RPT // L):
        zbuf[pl.ds(j * L, L)] = jnp.zeros((L,), jnp.float32)
    pltpu.sync_copy(zbuf, acc.at[pl.ds(sid * RPT, RPT)])

    pltpu.sync_copy(rowp2.at[pl.ds(wid * NCHUNK, NCHUNK)], r_all)
    pltpu.sync_copy(colp2.at[pl.ds(wid * NCHUNK, NCHUNK)], c_all)

    def doc(k, carry):
        for j in range(CHUNK // L):
            sl = pl.ds(j * L, L)
            r = r_all[k, sl]
            c = c_all[k, sl]
            c_all[k, sl] = jnp.where(r == c, jnp.full((L,), DUMMY, jnp.int32),
                                     c)
        return carry

    lax.fori_loop(0, NCHUNK, doc, 0)
    plsc.subcore_barrier()

    def step(k, carry):
        pltpu.async_copy(ones_b, acc.at[c_all.at[k]], sem, add=True)

        @pl.when(k >= DEG_WIN)
        def _():
            pltpu.make_async_copy(ones_b, acc.at[c_all.at[k - DEG_WIN]],
                                  sem).wait()

        return carry

    lax.fori_loop(0, NCHUNK, step, 0)
    for i in range(DEG_WIN):
        pltpu.make_async_copy(ones_b, acc.at[c_all.at[NCHUNK - DEG_WIN + i]],
                              sem).wait()
    plsc.subcore_barrier()
    pltpu.sync_copy(acc.at[pl.ds(sid * RPT, RPT)],
                    out.at[core, pl.ds(sid * RPT, RPT)])


_deg_kernel = functools.partial(
    pl.kernel,
    out_type=jax.ShapeDtypeStruct((NC, N_ACC), jnp.float32),
    mesh=plsc.VectorSubcoreMesh(core_axis_name="c", subcore_axis_name="s"),
    scratch_types=[
        pltpu.VMEM((NCHUNK, CHUNK), jnp.int32),
        pltpu.VMEM((NCHUNK, CHUNK), jnp.int32),
        pltpu.VMEM((CHUNK,), jnp.float32),
        pltpu.VMEM((RPT,), jnp.float32),
        pltpu.VMEM_SHARED((N_ACC,), jnp.float32),
        pltpu.SemaphoreType.DMA,
    ],
)(_deg_body)


def _edge_body(rowp2, colp2, g, out, r_grp, c_grp, vals, zbuf, acc,
               si, sg0, sg1, ss0, ss1):
    core = lax.axis_index("c")
    sid = lax.axis_index("s")
    wid = core * NS + sid
    sgs = (sg0, sg1)
    sss = (ss0, ss1)

    for i in range(16):
        for j in range(D // L):
            zbuf[i, pl.ds(j * L, L)] = jnp.zeros((L,), jnp.float32)

    def zstep(k, carry):
        pltpu.sync_copy(zbuf, acc.at[pl.ds(sid * RPT + k * 16, 16)])
        return carry

    lax.fori_loop(0, RPT // 16, zstep, 0)

    pltpu.sync_copy(rowp2.at[pl.ds(wid * NCHUNK, G)], r_grp.at[0])
    pltpu.sync_copy(colp2.at[pl.ds(wid * NCHUNK, G)], c_grp.at[0])
    _doctor_group(r_grp, c_grp, 0)
    plsc.subcore_barrier()

    def visit(kk, s, gb, first=False, start_next=True):
        t = 1 - s
        pltpu.make_async_copy(g.at[r_grp.at[gb, kk]], vals.at[s],
                              sgs[s]).wait()
        pltpu.async_copy(vals.at[s], acc.at[c_grp.at[gb, kk]], sss[s],
                         add=True)
        if not first:
            pltpu.make_async_copy(vals.at[t], acc.at[c_grp.at[0, 0]],
                                  sss[t]).wait()
        if start_next:
            pltpu.async_copy(g.at[r_grp.at[gb, kk + 1]], vals.at[t], sgs[t])

    pltpu.async_copy(g.at[r_grp.at[0, 0]], vals.at[0], sgs[0])

    for m in range(NG):
        gb = m % 2
        nb = (m + 1) % 2
        base = wid * NCHUNK + (m + 1) * G
        visit(0, 0, gb, first=(m == 0))
        visit(1, 1, gb)
        if m + 1 < NG:
            pltpu.async_copy(rowp2.at[pl.ds(base, G)], r_grp.at[nb], si)
            pltpu.async_copy(colp2.at[pl.ds(base, G)], c_grp.at[nb], si)

        def mid(kk2, carry):
            visit(kk2 * 2, 0, gb)
            visit(kk2 * 2 + 1, 1, gb)
            return carry

        lax.fori_loop(1, G // 2 - 1, mid, 0)
        visit(G - 2, 0, gb)
        visit(G - 1, 1, gb, start_next=False)
        if m + 1 < NG:
            pltpu.make_async_copy(rowp2.at[pl.ds(base, G)], r_grp.at[nb],
                                  si).wait()
            pltpu.make_async_copy(colp2.at[pl.ds(base, G)], c_grp.at[nb],
                                  si).wait()
            _doctor_group(r_grp, c_grp, nb)
            pltpu.async_copy(g.at[r_grp.at[nb, 0]], vals.at[0], sgs[0])

    pltpu.make_async_copy(vals.at[1], acc.at[c_grp.at[0, 0]], sss[1]).wait()
    plsc.subcore_barrier()
    pltpu.sync_copy(acc.at[pl.ds(sid * RPT, RPT)],
                    out.at[core, pl.ds(sid * RPT, RPT)])


_edge_kernel = functools.partial(
    pl.kernel,
    out_type=jax.ShapeDtypeStruct((NC, N_ACC, D), jnp.float32),
    mesh=plsc.VectorSubcoreMesh(core_axis_name="c", subcore_axis_name="s"),
    scratch_types=[
        pltpu.VMEM((2, G, CHUNK), jnp.int32),
        pltpu.VMEM((2, G, CHUNK), jnp.int32),
        pltpu.VMEM((2, CHUNK, D), jnp.float32),
        pltpu.VMEM((16, D), jnp.float32),
        pltpu.VMEM_SHARED((N_ACC, D), jnp.float32),
        pltpu.SemaphoreType.DMA,
        pltpu.SemaphoreType.DMA,
        pltpu.SemaphoreType.DMA,
        pltpu.SemaphoreType.DMA,
        pltpu.SemaphoreType.DMA,
    ],
)(_edge_body)


def _mm_body(x_ref, w_ref, deg_ref, g_ref):
    deg = deg_ref[0, :] + deg_ref[1, :] + 1.0
    dis = lax.rsqrt(deg)
    h = lax.dot_general(x_ref[...], w_ref[...], (((1,), (1,)), ((), ())),
                        preferred_element_type=jnp.float32)
    g_ref[...] = h * dis[:, None]


def _comb_body(p_ref, g_ref, deg_ref, b_ref, o_ref):
    deg = deg_ref[0, :] + deg_ref[1, :] + 1.0
    dis = lax.rsqrt(deg)
    s = p_ref[0] + p_ref[1] + g_ref[...]
    o_ref[...] = s * dis[:, None] + b_ref[...]


def kernel(x, edge_index, adj_norm_sp, W, bias):
    row = edge_index[0]
    col = edge_index[1]
    pad = E_PAD - E
    rowp2 = jnp.concatenate([row, jnp.zeros((pad,), jnp.int32)]).reshape(
        E_PAD // CHUNK, CHUNK)
    colp2 = jnp.concatenate([col, jnp.full((pad,), DUMMY, jnp.int32)]).reshape(
        E_PAD // CHUNK, CHUNK)
    x_pad = jnp.concatenate(
        [x, jnp.zeros((N_ACC - N_NODES, D), jnp.float32)], axis=0)
    bias2d = bias.reshape(1, D)

    degp = _deg_kernel(rowp2, colp2)

    g = pl.pallas_call(
        _mm_body,
        grid=(GRID,),
        in_specs=[
            pl.BlockSpec((BLK, D), lambda i: (i, 0)),
            pl.BlockSpec((D, D), lambda i: (0, 0)),
            pl.BlockSpec((NC, BLK), lambda i: (0, i)),
        ],
        out_specs=pl.BlockSpec((BLK, D), lambda i: (i, 0)),
        out_shape=jax.ShapeDtypeStruct((N_ACC, D), jnp.float32),
    )(x_pad, W, degp)

    parts = _edge_kernel(rowp2, colp2, g)

    out = pl.pallas_call(
        _comb_body,
        grid=(GRID,),
        in_specs=[
            pl.BlockSpec((NC, BLK, D), lambda i: (0, i, 0)),
            pl.BlockSpec((BLK, D), lambda i: (i, 0)),
            pl.BlockSpec((NC, BLK), lambda i: (0, i)),
            pl.BlockSpec((1, D), lambda i: (0, 0)),
        ],
        out_specs=pl.BlockSpec((BLK, D), lambda i: (i, 0)),
        out_shape=jax.ShapeDtypeStruct((N_ACC, D), jnp.float32),
    )(parts, g, degp, bias2d)

    return out[:N_NODES]


# speedup vs baseline: 39.4096x; 2.6414x over previous
import functools

import jax
import jax.numpy as jnp
from jax import lax
from jax.experimental import pallas as pl
from jax.experimental.pallas import tpu as pltpu
from jax.experimental.pallas import tpu_sc as plsc

N_NODES = 10000
D = 128
E = 320000

NC = 2
NS = 16
NW = NC * NS
L = 16

CHUNK = 128
NCHUNK = 80
EPW = CHUNK * NCHUNK
E_PAD = EPW * NW
N_ACC = 10240
DUMMY = N_NODES
RPT = N_ACC // NS
GRID = 10
BLK = N_ACC // GRID
G = 16
NG = NCHUNK // G
DEG_WIN = 8


def _doctor_group(r_grp, c_grp, gb):
    def doc(k, carry):
        for j in range(CHUNK // L):
            sl = pl.ds(j * L, L)
            r = r_grp[gb, k, sl]
            c = c_grp[gb, k, sl]
            c_grp[gb, k, sl] = jnp.where(
                r == c, jnp.full((L,), DUMMY, jnp.int32) + (c & 127), c)
        return carry

    lax.fori_loop(0, G, doc, 0)


def _deg_body(rowp2, colp2, out, r_all, c_all, ones_b, zbuf, acc, sem):
    core = lax.axis_index("c")
    sid = lax.axis_index("s")
    wid = core * NS + sid

    for j in range(CHUNK // L):
        ones_b[pl.ds(j * L, L)] = jnp.full((L,), 1.0, jnp.float32)
    for j in range(RPT // L):
        zbuf[pl.ds(j * L, L)] = jnp.zeros((L,), jnp.float32)
    pltpu.sync_copy(zbuf, acc.at[pl.ds(sid * RPT, RPT)])

    pltpu.sync_copy(rowp2.at[pl.ds(wid * NCHUNK, NCHUNK)], r_all)
    pltpu.sync_copy(colp2.at[pl.ds(wid * NCHUNK, NCHUNK)], c_all)

    def doc(k, carry):
        for j in range(CHUNK // L):
            sl = pl.ds(j * L, L)
            r = r_all[k, sl]
            c = c_all[k, sl]
            c_all[k, sl] = jnp.where(
                r == c, jnp.full((L,), DUMMY, jnp.int32) + (c & 127), c)
        return carry

    lax.fori_loop(0, NCHUNK, doc, 0)
    plsc.subcore_barrier()

    def step(k, carry):
        pltpu.async_copy(ones_b, acc.at[c_all.at[k]], sem, add=True)

        @pl.when(k >= DEG_WIN)
        def _():
            pltpu.make_async_copy(ones_b, acc.at[c_all.at[k - DEG_WIN]],
                                  sem).wait()

        return carry

    lax.fori_loop(0, NCHUNK, step, 0)
    for i in range(DEG_WIN):
        pltpu.make_async_copy(ones_b, acc.at[c_all.at[NCHUNK - DEG_WIN + i]],
                              sem).wait()
    plsc.subcore_barrier()
    pltpu.sync_copy(acc.at[pl.ds(sid * RPT, RPT)],
                    out.at[core, pl.ds(sid * RPT, RPT)])


_deg_kernel = functools.partial(
    pl.kernel,
    out_type=jax.ShapeDtypeStruct((NC, N_ACC), jnp.float32),
    mesh=plsc.VectorSubcoreMesh(core_axis_name="c", subcore_axis_name="s"),
    scratch_types=[
        pltpu.VMEM((NCHUNK, CHUNK), jnp.int32),
        pltpu.VMEM((NCHUNK, CHUNK), jnp.int32),
        pltpu.VMEM((CHUNK,), jnp.float32),
        pltpu.VMEM((RPT,), jnp.float32),
        pltpu.VMEM_SHARED((N_ACC,), jnp.float32),
        pltpu.SemaphoreType.DMA,
    ],
)(_deg_body)


def _edge_body(rowp2, colp2, g, out, r_grp, c_grp, vals, zbuf, acc,
               si, sg0, sg1, ss0, ss1):
    core = lax.axis_index("c")
    sid = lax.axis_index("s")
    wid = core * NS + sid
    sgs = (sg0, sg1)
    sss = (ss0, ss1)

    for i in range(16):
        for j in range(D // L):
            zbuf[i, pl.ds(j * L, L)] = jnp.zeros((L,), jnp.float32)

    def zstep(k, carry):
        pltpu.sync_copy(zbuf, acc.at[pl.ds(sid * RPT + k * 16, 16)])
        return carry

    lax.fori_loop(0, RPT // 16, zstep, 0)

    pltpu.sync_copy(rowp2.at[pl.ds(wid * NCHUNK, G)], r_grp.at[0])
    pltpu.sync_copy(colp2.at[pl.ds(wid * NCHUNK, G)], c_grp.at[0])
    _doctor_group(r_grp, c_grp, 0)
    plsc.subcore_barrier()

    def visit(kk, s, gb, first=False, start_next=True):
        t = 1 - s
        pltpu.make_async_copy(g.at[r_grp.at[gb, kk]], vals.at[s],
                              sgs[s]).wait()
        pltpu.async_copy(vals.at[s], acc.at[c_grp.at[gb, kk]], sss[s],
                         add=True)
        if not first:
            pltpu.make_async_copy(vals.at[t], acc.at[c_grp.at[0, 0]],
                                  sss[t]).wait()
        if start_next:
            pltpu.async_copy(g.at[r_grp.at[gb, kk + 1]], vals.at[t], sgs[t])

    pltpu.async_copy(g.at[r_grp.at[0, 0]], vals.at[0], sgs[0])

    for m in range(NG):
        gb = m % 2
        nb = (m + 1) % 2
        base = wid * NCHUNK + (m + 1) * G
        visit(0, 0, gb, first=(m == 0))
        visit(1, 1, gb)
        if m + 1 < NG:
            pltpu.async_copy(rowp2.at[pl.ds(base, G)], r_grp.at[nb], si)
            pltpu.async_copy(colp2.at[pl.ds(base, G)], c_grp.at[nb], si)

        def mid(kk2, carry):
            visit(kk2 * 2, 0, gb)
            visit(kk2 * 2 + 1, 1, gb)
            return carry

        lax.fori_loop(1, G // 2 - 1, mid, 0)
        visit(G - 2, 0, gb)
        visit(G - 1, 1, gb, start_next=False)
        if m + 1 < NG:
            pltpu.make_async_copy(rowp2.at[pl.ds(base, G)], r_grp.at[nb],
                                  si).wait()
            pltpu.make_async_copy(colp2.at[pl.ds(base, G)], c_grp.at[nb],
                                  si).wait()
            _doctor_group(r_grp, c_grp, nb)
            pltpu.async_copy(g.at[r_grp.at[nb, 0]], vals.at[0], sgs[0])

    pltpu.make_async_copy(vals.at[1], acc.at[c_grp.at[0, 0]], sss[1]).wait()
    plsc.subcore_barrier()
    pltpu.sync_copy(acc.at[pl.ds(sid * RPT, RPT)],
                    out.at[core, pl.ds(sid * RPT, RPT)])


_edge_kernel = functools.partial(
    pl.kernel,
    out_type=jax.ShapeDtypeStruct((NC, N_ACC, D), jnp.float32),
    mesh=plsc.VectorSubcoreMesh(core_axis_name="c", subcore_axis_name="s"),
    scratch_types=[
        pltpu.VMEM((2, G, CHUNK), jnp.int32),
        pltpu.VMEM((2, G, CHUNK), jnp.int32),
        pltpu.VMEM((2, CHUNK, D), jnp.float32),
        pltpu.VMEM((16, D), jnp.float32),
        pltpu.VMEM_SHARED((N_ACC, D), jnp.float32),
        pltpu.SemaphoreType.DMA,
        pltpu.SemaphoreType.DMA,
        pltpu.SemaphoreType.DMA,
        pltpu.SemaphoreType.DMA,
        pltpu.SemaphoreType.DMA,
    ],
)(_edge_body)


def _mm_body(x_ref, w_ref, deg_ref, g_ref):
    deg = deg_ref[0, :] + deg_ref[1, :] + 1.0
    dis = lax.rsqrt(deg)
    h = lax.dot_general(x_ref[...], w_ref[...], (((1,), (1,)), ((), ())),
                        preferred_element_type=jnp.float32)
    g_ref[...] = h * dis[:, None]


def _comb_body(p_ref, g_ref, deg_ref, b_ref, o_ref):
    deg = deg_ref[0, :] + deg_ref[1, :] + 1.0
    dis = lax.rsqrt(deg)
    s = p_ref[0] + p_ref[1] + g_ref[...]
    o_ref[...] = s * dis[:, None] + b_ref[...]


def kernel(x, edge_index, adj_norm_sp, W, bias):
    row = edge_index[0]
    col = edge_index[1]
    pad = E_PAD - E
    pad_idx = jnp.arange(pad, dtype=jnp.int32)
    rowp2 = jnp.concatenate([row, pad_idx % N_NODES]).reshape(
        E_PAD // CHUNK, CHUNK)
    colp2 = jnp.concatenate(
        [col, DUMMY + (pad_idx % (N_ACC - N_NODES))]).reshape(
        E_PAD // CHUNK, CHUNK)
    x_pad = jnp.concatenate(
        [x, jnp.zeros((N_ACC - N_NODES, D), jnp.float32)], axis=0)
    bias2d = bias.reshape(1, D)

    degp = _deg_kernel(rowp2, colp2)

    g = pl.pallas_call(
        _mm_body,
        grid=(GRID,),
        in_specs=[
            pl.BlockSpec((BLK, D), lambda i: (i, 0)),
            pl.BlockSpec((D, D), lambda i: (0, 0)),
            pl.BlockSpec((NC, BLK), lambda i: (0, i)),
        ],
        out_specs=pl.BlockSpec((BLK, D), lambda i: (i, 0)),
        out_shape=jax.ShapeDtypeStruct((N_ACC, D), jnp.float32),
    )(x_pad, W, degp)

    parts = _edge_kernel(rowp2, colp2, g)

    out = pl.pallas_call(
        _comb_body,
        grid=(GRID,),
        in_specs=[
            pl.BlockSpec((NC, BLK, D), lambda i: (0, i, 0)),
            pl.BlockSpec((BLK, D), lambda i: (i, 0)),
            pl.BlockSpec((NC, BLK), lambda i: (0, i)),
            pl.BlockSpec((1, D), lambda i: (0, 0)),
        ],
        out_specs=pl.BlockSpec((BLK, D), lambda i: (i, 0)),
        out_shape=jax.ShapeDtypeStruct((N_ACC, D), jnp.float32),
    )(parts, g, degp, bias2d)

    return out[:N_NODES]


# speedup vs baseline: 39.4590x; 1.0013x over previous
import functools

import jax
import jax.numpy as jnp
from jax import lax
from jax.experimental import pallas as pl
from jax.experimental.pallas import tpu as pltpu
from jax.experimental.pallas import tpu_sc as plsc

N_NODES = 10000
D = 128
E = 320000

NC = 2
NS = 16
NW = NC * NS
L = 16

CHUNK = 64
NCHUNK = 160
EPW = CHUNK * NCHUNK
E_PAD = EPW * NW
N_ACC = 10240
DUMMY = N_NODES
RPT = N_ACC // NS
GRID = 10
BLK = N_ACC // GRID
G = 16
NG = NCHUNK // G
DEG_WIN = 8


def _doctor_group(r_grp, c_grp, gb):
    def doc(k, carry):
        for j in range(CHUNK // L):
            sl = pl.ds(j * L, L)
            r = r_grp[gb, k, sl]
            c = c_grp[gb, k, sl]
            c_grp[gb, k, sl] = jnp.where(
                r == c, jnp.full((L,), DUMMY, jnp.int32) + (c & 127), c)
        return carry

    lax.fori_loop(0, G, doc, 0)


def _deg_body(rowp2, colp2, out, r_all, c_all, ones_b, zbuf, acc, sem):
    core = lax.axis_index("c")
    sid = lax.axis_index("s")
    wid = core * NS + sid

    for j in range(CHUNK // L):
        ones_b[pl.ds(j * L, L)] = jnp.full((L,), 1.0, jnp.float32)
    for j in range(RPT // L):
        zbuf[pl.ds(j * L, L)] = jnp.zeros((L,), jnp.float32)
    pltpu.sync_copy(zbuf, acc.at[pl.ds(sid * RPT, RPT)])

    pltpu.sync_copy(rowp2.at[pl.ds(wid * NCHUNK, NCHUNK)], r_all)
    pltpu.sync_copy(colp2.at[pl.ds(wid * NCHUNK, NCHUNK)], c_all)

    def doc(k, carry):
        for j in range(CHUNK // L):
            sl = pl.ds(j * L, L)
            r = r_all[k, sl]
            c = c_all[k, sl]
            c_all[k, sl] = jnp.where(
                r == c, jnp.full((L,), DUMMY, jnp.int32) + (c & 127), c)
        return carry

    lax.fori_loop(0, NCHUNK, doc, 0)
    plsc.subcore_barrier()

    def step(k, carry):
        pltpu.async_copy(ones_b, acc.at[c_all.at[k]], sem, add=True)

        @pl.when(k >= DEG_WIN)
        def _():
            pltpu.make_async_copy(ones_b, acc.at[c_all.at[k - DEG_WIN]],
                                  sem).wait()

        return carry

    lax.fori_loop(0, NCHUNK, step, 0)
    for i in range(DEG_WIN):
        pltpu.make_async_copy(ones_b, acc.at[c_all.at[NCHUNK - DEG_WIN + i]],
                              sem).wait()
    plsc.subcore_barrier()
    pltpu.sync_copy(acc.at[pl.ds(sid * RPT, RPT)],
                    out.at[core, pl.ds(sid * RPT, RPT)])


_deg_kernel = functools.partial(
    pl.kernel,
    out_type=jax.ShapeDtypeStruct((NC, N_ACC), jnp.float32),
    mesh=plsc.VectorSubcoreMesh(core_axis_name="c", subcore_axis_name="s"),
    scratch_types=[
        pltpu.VMEM((NCHUNK, CHUNK), jnp.int32),
        pltpu.VMEM((NCHUNK, CHUNK), jnp.int32),
        pltpu.VMEM((CHUNK,), jnp.float32),
        pltpu.VMEM((RPT,), jnp.float32),
        pltpu.VMEM_SHARED((N_ACC,), jnp.float32),
        pltpu.SemaphoreType.DMA,
    ],
)(_deg_body)


def _edge_body(rowp2, colp2, g, out, r_grp, c_grp, vals, zbuf, acc,
               si, sg0, sg1, sg2, sg3, ss0, ss1, ss2, ss3):
    core = lax.axis_index("c")
    sid = lax.axis_index("s")
    wid = core * NS + sid
    sgs = (sg0, sg1, sg2, sg3)
    sss = (ss0, ss1, ss2, ss3)

    for i in range(16):
        for j in range(D // L):
            zbuf[i, pl.ds(j * L, L)] = jnp.zeros((L,), jnp.float32)

    def zstep(k, carry):
        pltpu.sync_copy(zbuf, acc.at[pl.ds(sid * RPT + k * 16, 16)])
        return carry

    lax.fori_loop(0, RPT // 16, zstep, 0)

    pltpu.sync_copy(rowp2.at[pl.ds(wid * NCHUNK, G)], r_grp.at[0])
    pltpu.sync_copy(colp2.at[pl.ds(wid * NCHUNK, G)], c_grp.at[0])
    _doctor_group(r_grp, c_grp, 0)
    plsc.subcore_barrier()

    def visit(kk, s, gb, first=False, start_next=True):
        t = (s + 2) % 4
        pltpu.make_async_copy(g.at[r_grp.at[gb, kk]], vals.at[s],
                              sgs[s]).wait()
        pltpu.async_copy(vals.at[s], acc.at[c_grp.at[gb, kk]], sss[s],
                         add=True)
        if not first:
            pltpu.make_async_copy(vals.at[t], acc.at[c_grp.at[0, 0]],
                                  sss[t]).wait()
        if start_next:
            pltpu.async_copy(g.at[r_grp.at[gb, kk + 2]], vals.at[t], sgs[t])

    pltpu.async_copy(g.at[r_grp.at[0, 0]], vals.at[0], sgs[0])
    pltpu.async_copy(g.at[r_grp.at[0, 1]], vals.at[1], sgs[1])

    for m in range(NG):
        gb = m % 2
        nb = (m + 1) % 2
        base = wid * NCHUNK + (m + 1) * G
        visit(0, 0, gb, first=(m == 0))
        visit(1, 1, gb, first=(m == 0))
        if m + 1 < NG:
            pltpu.async_copy(rowp2.at[pl.ds(base, G)], r_grp.at[nb], si)
            pltpu.async_copy(colp2.at[pl.ds(base, G)], c_grp.at[nb], si)
        visit(2, 2, gb)
        visit(3, 3, gb)

        def mid(kk4, carry):
            for s in range(4):
                visit(kk4 * 4 + s, s, gb)
            return carry

        lax.fori_loop(1, G // 4 - 1, mid, 0)
        visit(G - 4, 0, gb)
        visit(G - 3, 1, gb)
        visit(G - 2, 2, gb, start_next=False)
        visit(G - 1, 3, gb, start_next=False)
        if m + 1 < NG:
            pltpu.make_async_copy(rowp2.at[pl.ds(base, G)], r_grp.at[nb],
                                  si).wait()
            pltpu.make_async_copy(colp2.at[pl.ds(base, G)], c_grp.at[nb],
                                  si).wait()
            _doctor_group(r_grp, c_grp, nb)
            pltpu.async_copy(g.at[r_grp.at[nb, 0]], vals.at[0], sgs[0])
            pltpu.async_copy(g.at[r_grp.at[nb, 1]], vals.at[1], sgs[1])

    pltpu.make_async_copy(vals.at[2], acc.at[c_grp.at[0, 0]], sss[2]).wait()
    pltpu.make_async_copy(vals.at[3], acc.at[c_grp.at[0, 0]], sss[3]).wait()
    plsc.subcore_barrier()
    pltpu.sync_copy(acc.at[pl.ds(sid * RPT, RPT)],
                    out.at[core, pl.ds(sid * RPT, RPT)])


_edge_kernel = functools.partial(
    pl.kernel,
    out_type=jax.ShapeDtypeStruct((NC, N_ACC, D), jnp.float32),
    mesh=plsc.VectorSubcoreMesh(core_axis_name="c", subcore_axis_name="s"),
    scratch_types=[
        pltpu.VMEM((2, G, CHUNK), jnp.int32),
        pltpu.VMEM((2, G, CHUNK), jnp.int32),
        pltpu.VMEM((4, CHUNK, D), jnp.float32),
        pltpu.VMEM((16, D), jnp.float32),
        pltpu.VMEM_SHARED((N_ACC, D), jnp.float32),
        pltpu.SemaphoreType.DMA,
        pltpu.SemaphoreType.DMA,
        pltpu.SemaphoreType.DMA,
        pltpu.SemaphoreType.DMA,
        pltpu.SemaphoreType.DMA,
        pltpu.SemaphoreType.DMA,
        pltpu.SemaphoreType.DMA,
        pltpu.SemaphoreType.DMA,
        pltpu.SemaphoreType.DMA,
    ],
)(_edge_body)


def _mm_body(x_ref, w_ref, deg_ref, g_ref):
    deg = deg_ref[0, :] + deg_ref[1, :] + 1.0
    dis = lax.rsqrt(deg)
    h = lax.dot_general(x_ref[...], w_ref[...], (((1,), (1,)), ((), ())),
                        preferred_element_type=jnp.float32)
    g_ref[...] = h * dis[:, None]


def _comb_body(p_ref, g_ref, deg_ref, b_ref, o_ref):
    deg = deg_ref[0, :] + deg_ref[1, :] + 1.0
    dis = lax.rsqrt(deg)
    s = p_ref[0] + p_ref[1] + g_ref[...]
    o_ref[...] = s * dis[:, None] + b_ref[...]


def kernel(x, edge_index, adj_norm_sp, W, bias):
    row = edge_index[0]
    col = edge_index[1]
    pad = E_PAD - E
    pad_idx = jnp.arange(pad, dtype=jnp.int32)
    rowp2 = jnp.concatenate([row, pad_idx % N_NODES]).reshape(
        E_PAD // CHUNK, CHUNK)
    colp2 = jnp.concatenate(
        [col, DUMMY + (pad_idx % (N_ACC - N_NODES))]).reshape(
        E_PAD // CHUNK, CHUNK)
    x_pad = jnp.concatenate(
        [x, jnp.zeros((N_ACC - N_NODES, D), jnp.float32)], axis=0)
    bias2d = bias.reshape(1, D)

    degp = _deg_kernel(rowp2, colp2)

    g = pl.pallas_call(
        _mm_body,
        grid=(GRID,),
        in_specs=[
            pl.BlockSpec((BLK, D), lambda i: (i, 0)),
            pl.BlockSpec((D, D), lambda i: (0, 0)),
            pl.BlockSpec((NC, BLK), lambda i: (0, i)),
        ],
        out_specs=pl.BlockSpec((BLK, D), lambda i: (i, 0)),
        out_shape=jax.ShapeDtypeStruct((N_ACC, D), jnp.float32),
    )(x_pad, W, degp)

    parts = _edge_kernel(rowp2, colp2, g)

    out = pl.pallas_call(
        _comb_body,
        grid=(GRID,),
        in_specs=[
            pl.BlockSpec((NC, BLK, D), lambda i: (0, i, 0)),
            pl.BlockSpec((BLK, D), lambda i: (i, 0)),
            pl.BlockSpec((NC, BLK), lambda i: (0, i)),
            pl.BlockSpec((1, D), lambda i: (0, 0)),
        ],
        out_specs=pl.BlockSpec((BLK, D), lambda i: (i, 0)),
        out_shape=jax.ShapeDtypeStruct((N_ACC, D), jnp.float32),
    )(parts, g, degp, bias2d)

    return out[:N_NODES]
